# baseline (speedup 1.0000x reference)
import functools

import jax
import jax.numpy as jnp
from jax import lax
from jax.experimental import pallas as pl
from jax.experimental.pallas import tpu as pltpu
from jax.experimental.pallas import tpu_sc as plsc

_N_E = 1024
_D = 64
_COST = 0.25


def _vq_body(n_rows, tile, grid_h, base, finalize,
             x_hbm, e_ref, cnt_in, lsum_in,
             dist_ref, idx_ref, enc_ref, cnt_out, lsum_out,
             cnt_acc, loss_acc, x_buf, x_sem):
    i = pl.program_id(0)

    @pl.when(i == 0)
    def _init():
        cnt_acc[...] = cnt_in[...]
        loss_acc[0] = lsum_in[0]
        pltpu.make_async_copy(x_hbm.at[base], x_buf.at[0], x_sem.at[0]).start()

    @pl.when(i + 1 < grid_h)
    def _next():
        pltpu.make_async_copy(x_hbm.at[base + i + 1], x_buf.at[(i + 1) % 2],
                              x_sem.at[(i + 1) % 2]).start()

    slot = i % 2
    pltpu.make_async_copy(x_hbm.at[base + i], x_buf.at[slot],
                          x_sem.at[slot]).wait()
    x = x_buf[slot]
    e = e_ref[...]
    xsq = jnp.sum(x * x, axis=1, keepdims=True)
    esq = jnp.sum(e * e, axis=0, keepdims=True)
    mm = jax.lax.dot_general(x, e, (((1,), (0,)), ((), ())),
                             precision=jax.lax.Precision.DEFAULT)
    dist = (xsq - 2.0 * mm) + esq
    dist_ref[...] = dist

    m = jnp.min(dist, axis=1, keepdims=True)
    colf = jax.lax.broadcasted_iota(
        jnp.int32, (tile, _N_E), 1).astype(jnp.float32)
    idxf = jnp.min(jnp.where(dist == m, colf, 2048.0), axis=1, keepdims=True)
    idx_ref[0] = jnp.reshape(idxf.astype(jnp.int32), (1, tile))

    enc = jnp.where(colf == idxf, 1.0, 0.0).astype(jnp.float32)
    enc_ref[...] = enc

    cnt_acc[...] += jnp.sum(enc, axis=0, keepdims=True)
    loss_acc[0] += jnp.sum(m)

    @pl.when(i == grid_h - 1)
    def _fin():
        if not finalize:
            cnt_out[...] = cnt_acc[...]
            lsum_out[0] = loss_acc[0]
        else:
            total = loss_acc[0]
            lsum_out[0] = (1.0 + _COST) * (total / float(n_rows * _D))
            avg = cnt_acc[...] * (1.0 / float(n_rows))
            ent = jnp.sum(avg * jnp.log(avg + 1e-10))
            cnt_out[...] = jnp.broadcast_to(jnp.exp(-ent), (1, _N_E))


def _tc_half(x3, embeddings, cnt_in, lsum_in, dist_io, enc_io,
             n_rows, tile, half, finalize):
    grid_h = (n_rows // tile) // 2
    base = half * grid_h

    out_shapes = (
        jax.ShapeDtypeStruct((n_rows, _N_E), jnp.float32),
        jax.ShapeDtypeStruct((grid_h, 1, tile), jnp.int32),
        jax.ShapeDtypeStruct((n_rows, _N_E), jnp.float32),
        jax.ShapeDtypeStruct((1, _N_E), jnp.float32),
        jax.ShapeDtypeStruct((1,), jnp.float32),
    )
    args = [x3, embeddings, cnt_in, lsum_in]
    in_specs = [
        pl.BlockSpec(memory_space=pltpu.MemorySpace.HBM),
        pl.BlockSpec((_D, _N_E), lambda i: (0, 0)),
        pl.BlockSpec((1, _N_E), lambda i: (0, 0)),
        pl.BlockSpec(memory_space=pltpu.SMEM),
    ]
    aliases = {}
    if dist_io is not None:
        args += [dist_io, enc_io]
        in_specs += [pl.BlockSpec(memory_space=pltpu.MemorySpace.HBM)] * 2
        aliases = {4: 0, 5: 2}

    def _body(x_hbm, e_ref, cnt_in_ref, lsum_in_ref, *rest):
        rest = list(rest)
        if dist_io is not None:
            del rest[0:2]
        return _vq_body(n_rows, tile, grid_h, base, finalize,
                        x_hbm, e_ref, cnt_in_ref, lsum_in_ref, *rest)

    return pl.pallas_call(
        _body,
        grid=(grid_h,),
        in_specs=in_specs,
        out_specs=[
            pl.BlockSpec((tile, _N_E), lambda i, b=base: (b + i, 0)),
            pl.BlockSpec((1, 1, tile), lambda i: (i, 0, 0)),
            pl.BlockSpec((tile, _N_E), lambda i, b=base: (b + i, 0)),
            pl.BlockSpec((1, _N_E), lambda i: (0, 0)),
            pl.BlockSpec(memory_space=pltpu.SMEM),
        ],
        out_shape=out_shapes,
        scratch_shapes=[
            pltpu.VMEM((1, _N_E), jnp.float32),
            pltpu.SMEM((1,), jnp.float32),
            pltpu.VMEM((2, tile, _D), jnp.float32),
            pltpu.SemaphoreType.DMA((2,)),
        ],
        input_output_aliases=aliases,
    )(*args)


def _make_sc_gather(n_half):
    info = plsc.get_sparse_core_info()
    nc, ns = info.num_cores, info.num_subcores
    nw = nc * ns
    bpw = n_half // nw
    chunks = [128] * (bpw // 128)
    if bpw % 128:
        chunks.append(bpw % 128)
    mesh = plsc.VectorSubcoreMesh(core_axis_name="c", subcore_axis_name="s")

    @functools.partial(
        pl.kernel, mesh=mesh,
        out_type=jax.ShapeDtypeStruct((n_half, 128), jnp.float32),
        scratch_types=[
            pltpu.VMEM((bpw,), jnp.int32),
            pltpu.VMEM((bpw, 128), jnp.float32),
            pltpu.SemaphoreType.DMA,
        ],
    )
    def sc_gather(table_hbm, idx_hbm, out_hbm, idx_v, rows_v, sem):
        wid = lax.axis_index("s") * nc + lax.axis_index("c")
        base = wid * bpw
        pltpu.sync_copy(idx_hbm.at[pl.ds(base, bpw)], idx_v)
        copies = []
        off = 0
        for n in chunks:
            copies.append(pltpu.async_copy(
                table_hbm.at[idx_v.at[pl.ds(off, n)]],
                rows_v.at[pl.ds(off, n)], sem))
            off += n
        for c in copies:
            c.wait()
        pltpu.sync_copy(rows_v, out_hbm.at[pl.ds(base, bpw)])

    return sc_gather


def kernel(inputs, context, embeddings):
    del context
    b, s = inputs.shape[0], inputs.shape[1]
    n_rows = b * s
    tile = 2 * s
    grid = n_rows // tile
    half_rows = n_rows // 2
    x3 = jnp.reshape(inputs, (grid, tile, _D))

    table = jnp.swapaxes(embeddings, 0, 1)
    table128 = jnp.pad(table, ((0, 0), (0, 128 - _D)))
    sc_gather = _make_sc_gather(half_rows)

    cnt0 = jnp.zeros((1, _N_E), jnp.float32)
    lsum0 = jnp.zeros((1,), jnp.float32)
    dist_a, idx3a, enc_a, cnt_a, lsum_a = _tc_half(
        x3, embeddings, cnt0, lsum0, None, None, n_rows, tile, 0, False)
    qa = sc_gather(table128, jnp.reshape(idx3a, (half_rows,)))
    dist, idx3b, enc, perp2d, loss1 = _tc_half(
        x3, embeddings, cnt_a, lsum_a, dist_a, enc_a, n_rows, tile, 1, True)
    qb = sc_gather(table128, jnp.reshape(idx3b, (half_rows,)))

    quantized = jnp.reshape(
        jnp.concatenate([qa[:, :_D], qb[:, :_D]], axis=0), inputs.shape)
    encoding_indices = jnp.reshape(
        jnp.concatenate([idx3a, idx3b], axis=0), (b, s))
    loss = jnp.reshape(loss1, ())
    perplexity = jnp.reshape(perp2d[0, 0], ())
    return (quantized, loss, perplexity, enc, encoding_indices, dist)


# speedup vs baseline: 1.2915x; 1.2915x over previous
import functools

import jax
import jax.numpy as jnp
from jax import lax
from jax.experimental import pallas as pl
from jax.experimental.pallas import tpu as pltpu
from jax.experimental.pallas import tpu_sc as plsc

_N_E = 1024
_D = 64
_COST = 0.25


def _vq_body(n_rows, tile, sub, grid,
             x_hbm, e_ref,
             dist_ref, idx_ref, enc_ref, loss_ref, perp_ref,
             cnt_acc, loss_acc, x_buf, x_sem):
    i = pl.program_id(0)

    @pl.when(i == 0)
    def _init():
        cnt_acc[...] = jnp.zeros_like(cnt_acc)
        loss_acc[0] = 0.0
        pltpu.make_async_copy(x_hbm.at[0], x_buf.at[0], x_sem.at[0]).start()

    @pl.when(i + 1 < grid)
    def _next():
        pltpu.make_async_copy(x_hbm.at[i + 1], x_buf.at[(i + 1) % 2],
                              x_sem.at[(i + 1) % 2]).start()

    slot = i % 2
    pltpu.make_async_copy(x_hbm.at[i], x_buf.at[slot], x_sem.at[slot]).wait()
    e = e_ref[...]
    esq = jnp.sum(e * e, axis=0, keepdims=True)
    colf = jax.lax.broadcasted_iota(
        jnp.int32, (sub, _N_E), 1).astype(jnp.float32)
    for j in range(tile // sub):
        xt = x_buf[slot, j]
        xsq = jnp.reshape(jnp.sum(xt * xt, axis=0, keepdims=True), (sub, 1))
        mm = jax.lax.dot_general(xt, e, (((0,), (0,)), ((), ())),
                                 precision=jax.lax.Precision.DEFAULT)
        dist = (xsq - 2.0 * mm) + esq
        dist_ref[pl.ds(j * sub, sub), :] = dist

        m = jnp.min(dist, axis=1, keepdims=True)
        idxf = jnp.min(jnp.where(dist == m, colf, 2048.0),
                       axis=1, keepdims=True)
        idx_ref[0, :, pl.ds(j * sub, sub)] = jnp.reshape(
            idxf.astype(jnp.int32), (1, sub))

        enc = jnp.where(colf == idxf, 1.0, 0.0).astype(jnp.float32)
        enc_ref[pl.ds(j * sub, sub), :] = enc

        cnt_acc[...] += jnp.sum(enc, axis=0, keepdims=True)
        loss_acc[0] += jnp.sum(m)

    @pl.when(i == grid - 1)
    def _fin():
        total = loss_acc[0]
        loss_ref[0, 0] = (1.0 + _COST) * (total / float(n_rows * _D))
        avg = cnt_acc[...] * (1.0 / float(n_rows))
        ent = jnp.sum(avg * jnp.log(avg + 1e-10))
        perp_ref[0, 0] = jnp.exp(-ent)


def _make_sc_gather(n_rows):
    info = plsc.get_sparse_core_info()
    nc, ns = info.num_cores, info.num_subcores
    nw = nc * ns
    bpw = n_rows // nw
    chunks = [128] * (bpw // 128)
    if bpw % 128:
        chunks.append(bpw % 128)
    mesh = plsc.VectorSubcoreMesh(core_axis_name="c", subcore_axis_name="s")

    @functools.partial(
        pl.kernel, mesh=mesh,
        out_type=jax.ShapeDtypeStruct((n_rows, 128), jnp.float32),
        scratch_types=[
            pltpu.VMEM((bpw,), jnp.int32),
            pltpu.VMEM((bpw, 128), jnp.float32),
            pltpu.SemaphoreType.DMA,
        ],
    )
    def sc_gather(table_hbm, idx_hbm, out_hbm, idx_v, rows_v, sem):
        wid = lax.axis_index("s") * nc + lax.axis_index("c")
        base = wid * bpw
        pltpu.sync_copy(idx_hbm.at[pl.ds(base, bpw)], idx_v)
        copies = []
        off = 0
        for n in chunks:
            copies.append(pltpu.async_copy(
                table_hbm.at[idx_v.at[pl.ds(off, n)]],
                rows_v.at[pl.ds(off, n)], sem))
            off += n
        for c in copies:
            c.wait()
        pltpu.sync_copy(rows_v, out_hbm.at[pl.ds(base, bpw)])

    return sc_gather


def kernel(inputs, context, embeddings):
    del context
    b, s = inputs.shape[0], inputs.shape[1]
    n_rows = b * s
    tile = 2 * s
    sub = s
    grid = n_rows // tile
    xt4 = jnp.reshape(jnp.swapaxes(inputs, 1, 2), (grid, 2, _D, s))

    out_shapes = (
        jax.ShapeDtypeStruct((n_rows, _N_E), jnp.float32),
        jax.ShapeDtypeStruct((grid, 1, tile), jnp.int32),
        jax.ShapeDtypeStruct((n_rows, _N_E), jnp.float32),
        jax.ShapeDtypeStruct((1, 1), jnp.float32),
        jax.ShapeDtypeStruct((1, 1), jnp.float32),
    )
    dist, idx3, enc, loss, perp = pl.pallas_call(
        functools.partial(_vq_body, n_rows, tile, sub, grid),
        grid=(grid,),
        in_specs=[
            pl.BlockSpec(memory_space=pltpu.MemorySpace.HBM),
            pl.BlockSpec((_D, _N_E), lambda i: (0, 0)),
        ],
        out_specs=[
            pl.BlockSpec((tile, _N_E), lambda i: (i, 0)),
            pl.BlockSpec((1, 1, tile), lambda i: (i, 0, 0)),
            pl.BlockSpec((tile, _N_E), lambda i: (i, 0)),
            pl.BlockSpec(memory_space=pltpu.SMEM),
            pl.BlockSpec(memory_space=pltpu.SMEM),
        ],
        out_shape=out_shapes,
        scratch_shapes=[
            pltpu.VMEM((1, _N_E), jnp.float32),
            pltpu.SMEM((1,), jnp.float32),
            pltpu.VMEM((2, 2, _D, sub), jnp.float32),
            pltpu.SemaphoreType.DMA((2,)),
        ],
    )(xt4, embeddings)

    idx_flat = jnp.reshape(idx3, (n_rows,))
    table = jnp.swapaxes(embeddings, 0, 1)
    table128 = jnp.pad(table, ((0, 0), (0, 128 - _D)))
    q128 = _make_sc_gather(n_rows)(table128, idx_flat)

    quantized = jnp.reshape(q128[:, :_D], inputs.shape)
    encoding_indices = jnp.reshape(idx3, (b, s))
    return (quantized, jnp.reshape(loss, ()), jnp.reshape(perp, ()),
            enc, encoding_indices, dist)
